# Initial kernel scaffold; baseline (speedup 1.0000x reference)
import jax
import jax.numpy as jnp
from jax.experimental import pallas as pl


def kernel(outputs, indices, memory_bank, W, b, neg_idxs):
    raise NotImplementedError("write your pallas kernel here")


# speedup vs baseline: 1.0314x; 1.0314x over previous
import functools

import jax
import jax.numpy as jnp
from jax import lax
from jax.experimental import pallas as pl
from jax.experimental.pallas import tpu as pltpu
from jax.experimental.pallas import tpu_sc as plsc

_N = 1000000
_D_OUT = 2048
_EMB = 128
_B = 1024
_M = 1024
_TAU = 0.07
_GAMMA = 0.5

_NW = 32
_CHUNK = 128
_NEG_PER_W = (_B * _M) // _NW
_NCHUNKS = _NEG_PER_W // _CHUNK
_POS_PER_W = _B // _NW


def _emb_body(x_ref, w_ref, b_ref, o_ref):
    e = jnp.dot(x_ref[...], w_ref[...], preferred_element_type=jnp.float32)
    e = e + b_ref[...]
    n = jnp.sqrt(jnp.sum(e * e, axis=1, keepdims=True))
    o_ref[...] = e / jnp.maximum(n, 1e-12)


def _emb(outputs, W, b2d):
    return pl.pallas_call(
        _emb_body,
        out_shape=jax.ShapeDtypeStruct((_B, _EMB), jnp.float32),
    )(outputs, W, b2d)


def _sc_gather(neg_idx, pos_idx, bank):
    mesh = plsc.VectorSubcoreMesh(core_axis_name="c", subcore_axis_name="s")

    @functools.partial(
        pl.kernel,
        mesh=mesh,
        out_type=[
            jax.ShapeDtypeStruct((_B * _M, _EMB), jnp.float32),
            jax.ShapeDtypeStruct((_B, _EMB), jnp.float32),
        ],
        scratch_types=[
            pltpu.VMEM((_CHUNK,), jnp.int32),
            pltpu.VMEM((_CHUNK, _EMB), jnp.float32),
            pltpu.VMEM((_POS_PER_W,), jnp.int32),
            pltpu.VMEM((_POS_PER_W, _EMB), jnp.float32),
            pltpu.SemaphoreType.DMA,
        ],
    )
    def k(neg_idx_hbm, pos_idx_hbm, bank_hbm, neg_out, pos_out,
          idx_v, rows_v, pidx_v, prows_v, sem):
        wid = lax.axis_index("s") * 2 + lax.axis_index("c")
        base = wid * _NEG_PER_W

        def body(j, carry):
            off = base + j * _CHUNK
            pltpu.sync_copy(neg_idx_hbm.at[pl.ds(off, _CHUNK)], idx_v)
            pltpu.async_copy(bank_hbm.at[idx_v], rows_v, sem).wait()
            pltpu.sync_copy(rows_v, neg_out.at[pl.ds(off, _CHUNK)])
            return carry

        lax.fori_loop(0, _NCHUNKS, body, 0)

        pbase = wid * _POS_PER_W
        pltpu.sync_copy(pos_idx_hbm.at[pl.ds(pbase, _POS_PER_W)], pidx_v)
        pltpu.async_copy(bank_hbm.at[pidx_v], prows_v, sem).wait()
        pltpu.sync_copy(prows_v, pos_out.at[pl.ds(pbase, _POS_PER_W)])

    return k(neg_idx, pos_idx, bank)


_BI = 16


def _loss_body(emb_ref, pos_ref, neg_ref, upd_ref, ds_ref, ns_ref):
    e = emb_ref[...]
    p = pos_ref[...]
    nm = neg_ref[...]

    u_pos = jnp.sum(e * p, axis=1, keepdims=True) / _TAU
    ni = lax.dot_general(e, nm, (((1,), (2,)), ((0,), (0,))),
                         preferred_element_type=jnp.float32)
    u_neg = ni / _TAU

    m = jnp.max(u_neg, axis=1, keepdims=True)
    log_c = m + jnp.log(jnp.sum(jnp.exp(u_neg - m), axis=1, keepdims=True))

    mx = jnp.maximum(u_pos, log_c)
    log_data_denom = mx + jnp.log(jnp.exp(u_pos - mx) + jnp.exp(log_c - mx))
    data_part = jnp.sum(u_pos - log_data_denom)

    mx2 = jnp.maximum(u_neg, log_c)
    log_noise_denom = mx2 + jnp.log(jnp.exp(u_neg - mx2) + jnp.exp(log_c - mx2))
    noise_part = jnp.sum(log_c - log_noise_denom)

    upd = _GAMMA * p + (1.0 - _GAMMA) * e
    nrm = jnp.sqrt(jnp.sum(upd * upd, axis=1, keepdims=True))
    upd_ref[...] = upd / jnp.maximum(nrm, 1e-12)

    @pl.when(pl.program_id(0) == 0)
    def _():
        ds_ref[0, 0] = 0.0
        ns_ref[0, 0] = 0.0

    ds_ref[0, 0] += data_part
    ns_ref[0, 0] += noise_part


def _loss(emb, pos_mem, neg_mem):
    grid = (_B // _BI,)
    return pl.pallas_call(
        _loss_body,
        grid=grid,
        in_specs=[
            pl.BlockSpec((_BI, _EMB), lambda i: (i, 0)),
            pl.BlockSpec((_BI, _EMB), lambda i: (i, 0)),
            pl.BlockSpec((_BI, _M, _EMB), lambda i: (i, 0, 0)),
        ],
        out_specs=[
            pl.BlockSpec((_BI, _EMB), lambda i: (i, 0)),
            pl.BlockSpec(memory_space=pltpu.SMEM),
            pl.BlockSpec(memory_space=pltpu.SMEM),
        ],
        out_shape=[
            jax.ShapeDtypeStruct((_B, _EMB), jnp.float32),
            jax.ShapeDtypeStruct((1, 1), jnp.float32),
            jax.ShapeDtypeStruct((1, 1), jnp.float32),
        ],
    )(emb, pos_mem, neg_mem)


def kernel(outputs, indices, memory_bank, W, b, neg_idxs):
    emb = _emb(outputs.astype(jnp.float32), W, b.reshape(1, _EMB))
    neg_flat, pos_mem = _sc_gather(
        neg_idxs.reshape(-1).astype(jnp.int32),
        indices.astype(jnp.int32),
        memory_bank,
    )
    updated, dsum, nsum = _loss(emb, pos_mem, neg_flat.reshape(_B, _M, _EMB))
    data_loss = -dsum[0, 0] / _B
    noise_loss = -nsum[0, 0] / _B
    loss = data_loss + noise_loss
    return loss, updated, data_loss, noise_loss
